# baseline (speedup 1.0000x reference)
import functools

import jax
import jax.numpy as jnp
from jax.experimental import pallas as pl
from jax.experimental.pallas import tpu as pltpu

_RG = 16
_MC = 1024


def _chamfer_body(
    x_ref, yt_ref, loss_ref, rmin_ref, ymin_ref, *, n_total, m_total, nt_steps, b_total, tn, m
):
    b = pl.program_id(0)
    nt = pl.program_id(1)
    inf = jnp.array(float("inf"), jnp.bfloat16)

    @pl.when(jnp.logical_and(b == 0, nt == 0))
    def _init_loss():
        loss_ref[0, 0] = 0.0

    @pl.when(nt == 0)
    def _init_ymin():
        ymin_ref[...] = jnp.full((_RG, m), inf, jnp.bfloat16)

    rmin_ref[...] = jnp.full((tn, 128), inf, jnp.bfloat16)

    for mc in range(m // _MC):
        y0 = yt_ref[0, 0:1, mc * _MC:(mc + 1) * _MC]
        y1 = yt_ref[0, 1:2, mc * _MC:(mc + 1) * _MC]
        y2 = yt_ref[0, 2:3, mc * _MC:(mc + 1) * _MC]

        def rg_body(rg, carry, y0=y0, y1=y1, y2=y2, mc=mc):
            xr = x_ref[0, pl.ds(rg * _RG, _RG), :]
            d = (
                jnp.abs(xr[:, 0:1] - y0)
                + jnp.abs(xr[:, 1:2] - y1)
                + jnp.abs(xr[:, 2:3] - y2)
            )
            ymin_ref[:, mc * _MC:(mc + 1) * _MC] = jnp.minimum(
                ymin_ref[:, mc * _MC:(mc + 1) * _MC], d
            )
            dm = d[:, 0:128]
            for k in range(1, _MC // 128):
                dm = jnp.minimum(dm, d[:, k * 128:(k + 1) * 128])
            rmin_ref[pl.ds(rg * _RG, _RG), :] = jnp.minimum(
                rmin_ref[pl.ds(rg * _RG, _RG), :], dm
            )
            return carry

        jax.lax.fori_loop(0, tn // _RG, rg_body, 0)

    sx = jnp.sum(jnp.min(rmin_ref[...], axis=1).astype(jnp.float32))
    loss_ref[0, 0] += sx / (n_total * b_total)

    @pl.when(nt == nt_steps - 1)
    def _finish_batch():
        ys = jnp.sum(jnp.min(ymin_ref[...], axis=0).astype(jnp.float32))
        loss_ref[0, 0] += ys / (m_total * b_total)


def kernel(mesh_x, mesh_y):
    B, N, D = mesh_x.shape
    _, M, _ = mesh_y.shape
    TN = 512
    NT = N // TN

    x_bf = mesh_x.astype(jnp.bfloat16)
    yt = jnp.transpose(mesh_y, (0, 2, 1)).astype(jnp.bfloat16)

    body = functools.partial(
        _chamfer_body,
        n_total=float(N),
        m_total=float(M),
        nt_steps=NT,
        b_total=float(B),
        tn=TN,
        m=M,
    )

    loss = pl.pallas_call(
        body,
        grid=(B, NT),
        in_specs=[
            pl.BlockSpec((1, TN, D), lambda b, nt: (b, nt, 0)),
            pl.BlockSpec((1, D, M), lambda b, nt: (b, 0, 0)),
        ],
        out_specs=pl.BlockSpec(
            (1, 1), lambda b, nt: (0, 0), memory_space=pltpu.SMEM
        ),
        out_shape=jax.ShapeDtypeStruct((1, 1), jnp.float32),
        scratch_shapes=[
            pltpu.VMEM((TN, 128), jnp.bfloat16),
            pltpu.VMEM((_RG, M), jnp.bfloat16),
        ],
    )(x_bf, yt)

    return loss[0, 0]


# speedup vs baseline: 4.9927x; 4.9927x over previous
import functools

import jax
import jax.numpy as jnp
from jax.experimental import pallas as pl
from jax.experimental.pallas import tpu as pltpu


def _chamfer_body(x_ref, yt_ref, loss_ref, ymin_ref, *, n_total, m_total, nt_steps, b_total):
    b = pl.program_id(0)
    nt = pl.program_id(1)

    x = x_ref[0]
    yt = yt_ref[0]

    d = (
        jnp.abs(x[:, 0:1] - yt[0:1, :])
        + jnp.abs(x[:, 1:2] - yt[1:2, :])
        + jnp.abs(x[:, 2:3] - yt[2:3, :])
    )

    sx = jnp.sum(jnp.min(d, axis=1).astype(jnp.float32))
    ym = jnp.min(d, axis=0, keepdims=True)

    @pl.when(jnp.logical_and(b == 0, nt == 0))
    def _init_loss():
        loss_ref[0, 0] = 0.0

    @pl.when(nt == 0)
    def _init_ymin():
        ymin_ref[...] = ym

    @pl.when(nt != 0)
    def _acc_ymin():
        ymin_ref[...] = jnp.minimum(ymin_ref[...], ym)

    loss_ref[0, 0] += sx / (n_total * b_total)

    @pl.when(nt == nt_steps - 1)
    def _finish_batch():
        loss_ref[0, 0] += jnp.sum(ymin_ref[...].astype(jnp.float32)) / (
            m_total * b_total
        )


def kernel(mesh_x, mesh_y):
    B, N, D = mesh_x.shape
    _, M, _ = mesh_y.shape
    TN = 1024
    NT = N // TN

    x_bf = mesh_x.astype(jnp.bfloat16)
    yt = jnp.transpose(mesh_y, (0, 2, 1)).astype(jnp.bfloat16)

    body = functools.partial(
        _chamfer_body,
        n_total=float(N),
        m_total=float(M),
        nt_steps=NT,
        b_total=float(B),
    )

    loss = pl.pallas_call(
        body,
        grid=(B, NT),
        in_specs=[
            pl.BlockSpec((1, TN, D), lambda b, nt: (b, nt, 0)),
            pl.BlockSpec((1, D, M), lambda b, nt: (b, 0, 0)),
        ],
        out_specs=pl.BlockSpec(
            (1, 1), lambda b, nt: (0, 0), memory_space=pltpu.SMEM
        ),
        out_shape=jax.ShapeDtypeStruct((1, 1), jnp.float32),
        scratch_shapes=[pltpu.VMEM((1, M), jnp.bfloat16)],
    )(x_bf, yt)

    return loss[0, 0]


# speedup vs baseline: 5.1056x; 1.0226x over previous
import functools

import jax
import jax.numpy as jnp
from jax.experimental import pallas as pl
from jax.experimental.pallas import tpu as pltpu


def _chamfer_body(x_ref, yt_ref, loss_ref, ymin_ref, *, n_total, m_total, nt_steps, b_total):
    b = pl.program_id(0)
    nt = pl.program_id(1)

    x = x_ref[0]
    yt = yt_ref[0]

    d = (
        jnp.abs(x[:, 0:1] - yt[0:1, :])
        + jnp.abs(x[:, 1:2] - yt[1:2, :])
        + jnp.abs(x[:, 2:3] - yt[2:3, :])
    )

    sx = jnp.sum(jnp.min(d, axis=1).astype(jnp.float32))
    ym = jnp.min(d, axis=0, keepdims=True)

    @pl.when(jnp.logical_and(b == 0, nt == 0))
    def _init_loss():
        loss_ref[0, 0] = 0.0

    @pl.when(nt == 0)
    def _init_ymin():
        ymin_ref[...] = ym

    @pl.when(nt != 0)
    def _acc_ymin():
        ymin_ref[...] = jnp.minimum(ymin_ref[...], ym)

    loss_ref[0, 0] += sx / (n_total * b_total)

    @pl.when(nt == nt_steps - 1)
    def _finish_batch():
        loss_ref[0, 0] += jnp.sum(ymin_ref[...].astype(jnp.float32)) / (
            m_total * b_total
        )


def kernel(mesh_x, mesh_y):
    B, N, D = mesh_x.shape
    _, M, _ = mesh_y.shape
    TN = 2048
    NT = N // TN

    x_bf = mesh_x.astype(jnp.bfloat16)
    yt = jnp.transpose(mesh_y, (0, 2, 1)).astype(jnp.bfloat16)

    body = functools.partial(
        _chamfer_body,
        n_total=float(N),
        m_total=float(M),
        nt_steps=NT,
        b_total=float(B),
    )

    loss = pl.pallas_call(
        body,
        grid=(B, NT),
        in_specs=[
            pl.BlockSpec((1, TN, D), lambda b, nt: (b, nt, 0)),
            pl.BlockSpec((1, D, M), lambda b, nt: (b, 0, 0)),
        ],
        out_specs=pl.BlockSpec(
            (1, 1), lambda b, nt: (0, 0), memory_space=pltpu.SMEM
        ),
        out_shape=jax.ShapeDtypeStruct((1, 1), jnp.float32),
        scratch_shapes=[pltpu.VMEM((1, M), jnp.bfloat16)],
    )(x_bf, yt)

    return loss[0, 0]
